# baseline (speedup 1.0000x reference)
import jax
import jax.numpy as jnp
from jax import lax
from jax.experimental import pallas as pl
from jax.experimental.pallas import tpu as pltpu
from jax.experimental.pallas import tpu_sc as plsc

SEQ = 9
DM = 1024
EPS = 1e-7
_NW = 32
_ROWS_W = 64
_CROWS = 16
_CHUNK = _CROWS * DM
_NSLABS = 18
_CPS = _ROWS_W // _CROWS
_NCHUNKS = _NSLABS * _CPS


def _renorm_into(table_hbm, enc_v, row_v, sq_v):
    def one_row(r, carry):
        pltpu.sync_copy(table_hbm.at[r], row_v)

        def sumsq(j, acc):
            v = row_v[pl.ds(j * 16, 16)]
            return acc + v * v

        acc = lax.fori_loop(0, DM // 16, sumsq, jnp.zeros((16,), jnp.float32))
        sq_v[...] = acc

        def lanesum(j, tot):
            return tot + plsc.load_gather(sq_v, [jnp.full((16,), j, jnp.int32)])

        nsq = lax.fori_loop(0, 16, lanesum, jnp.zeros((16,), jnp.float32))
        i = plsc.bitcast(nsq, jnp.int32)
        y = plsc.bitcast(jnp.int32(0x5F3759DF) - (i >> 1), jnp.float32)
        for _ in range(4):
            y = y * (1.5 - 0.5 * nsq * y * y)
        norm = nsq * y
        scale = jnp.where(norm > 1.0, 1.0 / (norm + EPS), jnp.float32(1.0))

        def scale_row(j, c):
            enc_v[pl.ds(r * DM + j * 16, 16)] = row_v[pl.ds(j * 16, 16)] * scale
            return c

        lax.fori_loop(0, DM // 16, scale_row, 0)
        return carry

    lax.fori_loop(0, SEQ, one_row, 0)


def _sc_body(x_hbm, table_hbm, out_hbm,
             enc_v, row_v, sq_v, ib0, ib1, ob0, ob1,
             is0, is1, os0, os1):
    wid = lax.axis_index("c") * 16 + lax.axis_index("s")
    _renorm_into(table_hbm, enc_v, row_v, sq_v)

    def offs(t):
        slab = t // _CPS
        c = t - slab * _CPS
        base = slab * 2048 + wid * _ROWS_W + c * _CROWS
        return base, (slab % SEQ) * DM

    o0, _ = offs(0)
    pltpu.make_async_copy(x_hbm.at[pl.ds(o0, _CROWS)], ib0, is0).start()
    o1, _ = offs(1)
    pltpu.make_async_copy(x_hbm.at[pl.ds(o1, _CROWS)], ib1, is1).start()

    bufs = ((ib0, ob0, is0, os0), (ib1, ob1, is1, os1))

    def outer2(t0, carry):
        for b, (ib, ob, isem, osem) in enumerate(bufs):
            t = t0 * 2 + b
            off, eoff = offs(t)
            pltpu.make_async_copy(x_hbm.at[pl.ds(off, _CROWS)], ib, isem).wait()

            @pl.when(t >= 2)
            def _():
                poff, _ = offs(t - 2)
                pltpu.make_async_copy(
                    ob, out_hbm.at[pl.ds(poff, _CROWS)], osem
                ).wait()

            def add_body(v, c):
                ev = enc_v[pl.ds(eoff + v * 16, 16)]
                for r in range(_CROWS):
                    sl = pl.ds(v * 16, 16)
                    ob[r, sl] = ib[r, sl] + ev
                return c

            lax.fori_loop(0, DM // 16, add_body, 0)
            pltpu.make_async_copy(ob, out_hbm.at[pl.ds(off, _CROWS)], osem).start()

            @pl.when(t + 2 < _NCHUNKS)
            def _():
                noff, _ = offs(t + 2)
                pltpu.make_async_copy(
                    x_hbm.at[pl.ds(noff, _CROWS)], ib, isem
                ).start()
        return carry

    lax.fori_loop(0, _NCHUNKS // 2, outer2, 0)

    po0, _ = offs(_NCHUNKS - 2)
    pltpu.make_async_copy(ob0, out_hbm.at[pl.ds(po0, _CROWS)], os0).wait()
    po1, _ = offs(_NCHUNKS - 1)
    pltpu.make_async_copy(ob1, out_hbm.at[pl.ds(po1, _CROWS)], os1).wait()


def kernel(x, table):
    b, s, n, d = x.shape
    xf = x.reshape(b * s * n, d)
    out = pl.kernel(
        _sc_body,
        out_type=jax.ShapeDtypeStruct((b * s * n, d), jnp.float32),
        mesh=plsc.VectorSubcoreMesh(core_axis_name="c", subcore_axis_name="s"),
        scratch_types=[
            pltpu.VMEM((SEQ * DM,), jnp.float32),
            pltpu.VMEM((DM,), jnp.float32),
            pltpu.VMEM((16,), jnp.float32),
            pltpu.VMEM((_CROWS, DM), jnp.float32),
            pltpu.VMEM((_CROWS, DM), jnp.float32),
            pltpu.VMEM((_CROWS, DM), jnp.float32),
            pltpu.VMEM((_CROWS, DM), jnp.float32),
            pltpu.SemaphoreType.DMA,
            pltpu.SemaphoreType.DMA,
            pltpu.SemaphoreType.DMA,
            pltpu.SemaphoreType.DMA,
        ],
        compiler_params=pltpu.CompilerParams(needs_layout_passes=False),
    )(xf, table)
    return out.reshape(b, s, n, d)


# speedup vs baseline: 1.4548x; 1.4548x over previous
import jax
import jax.numpy as jnp
from jax import lax
from jax.experimental import pallas as pl
from jax.experimental.pallas import tpu as pltpu
from jax.experimental.pallas import tpu_sc as plsc

SEQ = 9
DM = 1024
EPS = 1e-7
_NC = 2


def _renorm_body(table_hbm, enc_hbm, row_v, sq_v):
    wid = lax.axis_index("c") * 16 + lax.axis_index("s")

    @pl.when(wid < SEQ)
    def _():
        pltpu.sync_copy(table_hbm.at[wid], row_v)

        def sumsq(j, acc):
            v = row_v[pl.ds(j * 16, 16)]
            return acc + v * v

        acc = lax.fori_loop(0, DM // 16, sumsq, jnp.zeros((16,), jnp.float32))
        sq_v[...] = acc
        def lanesum(j, tot):
            return tot + plsc.load_gather(sq_v, [jnp.full((16,), j, jnp.int32)])

        nsq = lax.fori_loop(0, 16, lanesum, jnp.zeros((16,), jnp.float32))
        i = plsc.bitcast(nsq, jnp.int32)
        y = plsc.bitcast(jnp.int32(0x5F3759DF) - (i >> 1), jnp.float32)
        for _ in range(4):
            y = y * (1.5 - 0.5 * nsq * y * y)
        norm = nsq * y
        scale = jnp.where(norm > 1.0, 1.0 / (norm + EPS), jnp.float32(1.0))

        def scale_row(j, c):
            sl = pl.ds(j * 16, 16)
            row_v[sl] = row_v[sl] * scale
            return c

        lax.fori_loop(0, DM // 16, scale_row, 0)
        pltpu.sync_copy(row_v, enc_hbm.at[wid])


def _renorm_table_sc(table):
    return pl.kernel(
        _renorm_body,
        out_type=jax.ShapeDtypeStruct((SEQ, DM), jnp.float32),
        mesh=plsc.VectorSubcoreMesh(
            core_axis_name="c", subcore_axis_name="s", num_cores=1
        ),
        scratch_types=[
            pltpu.VMEM((DM,), jnp.float32),
            pltpu.VMEM((16,), jnp.float32),
        ],
        compiler_params=pltpu.CompilerParams(
            needs_layout_passes=False,
            skip_device_barrier=True,
        ),
    )(table)


def _add_enc_kernel(x_ref, row_ref, o_ref):
    o_ref[...] = x_ref[...] + row_ref[...]


def kernel(x, table):
    b, s, n, d = x.shape
    enc = _renorm_table_sc(table).reshape(SEQ, 1, d)
    xr = x.reshape(b * s, n, d)
    out = pl.pallas_call(
        _add_enc_kernel,
        grid=(b * s,),
        in_specs=[
            pl.BlockSpec((1, n, d), lambda i: (i, 0, 0)),
            pl.BlockSpec((1, 1, d), lambda i: (i % SEQ, 0, 0)),
        ],
        out_specs=pl.BlockSpec((1, n, d), lambda i: (i, 0, 0)),
        out_shape=jax.ShapeDtypeStruct((b * s, n, d), x.dtype),
        compiler_params=pltpu.CompilerParams(
            dimension_semantics=("arbitrary",),
            vmem_limit_bytes=60 * 1024 * 1024,
        ),
    )(xr, enc)
    return out.reshape(b, s, n, d)


# speedup vs baseline: 1.4730x; 1.0125x over previous
import jax
import jax.numpy as jnp
from jax import lax
from jax.experimental import pallas as pl
from jax.experimental.pallas import tpu as pltpu
from jax.experimental.pallas import tpu_sc as plsc

SEQ = 9
DM = 1024
EPS = 1e-7
_NC = 2


def _renorm_body(table_hbm, enc_hbm, row_v, sq_v):
    wid = lax.axis_index("c") * 16 + lax.axis_index("s")

    @pl.when(wid < SEQ)
    def _():
        pltpu.sync_copy(table_hbm.at[wid], row_v)

        def sumsq(j, acc):
            v = row_v[pl.ds(j * 16, 16)]
            return acc + v * v

        acc = lax.fori_loop(0, DM // 16, sumsq, jnp.zeros((16,), jnp.float32))
        sq_v[...] = acc
        def lanesum(j, tot):
            return tot + plsc.load_gather(sq_v, [jnp.full((16,), j, jnp.int32)])

        nsq = lax.fori_loop(0, 16, lanesum, jnp.zeros((16,), jnp.float32))
        i = plsc.bitcast(nsq, jnp.int32)
        y = plsc.bitcast(jnp.int32(0x5F3759DF) - (i >> 1), jnp.float32)
        for _ in range(4):
            y = y * (1.5 - 0.5 * nsq * y * y)
        norm = nsq * y
        scale = jnp.where(norm > 1.0, 1.0 / (norm + EPS), jnp.float32(1.0))

        def scale_row(j, c):
            sl = pl.ds(j * 16, 16)
            row_v[sl] = row_v[sl] * scale
            return c

        lax.fori_loop(0, DM // 16, scale_row, 0)
        pltpu.sync_copy(row_v, enc_hbm.at[wid])


def _renorm_table_sc(table):
    return pl.kernel(
        _renorm_body,
        out_type=jax.ShapeDtypeStruct((SEQ, DM), jnp.float32),
        mesh=plsc.VectorSubcoreMesh(
            core_axis_name="c", subcore_axis_name="s", num_cores=1
        ),
        scratch_types=[
            pltpu.VMEM((DM,), jnp.float32),
            pltpu.VMEM((16,), jnp.float32),
        ],
        compiler_params=pltpu.CompilerParams(
            needs_layout_passes=False,
            skip_device_barrier=True,
        ),
    )(table)


def _add_enc_kernel(x_ref, enc_ref, o_ref):
    sid = lax.rem(pl.program_id(0), SEQ)
    row = enc_ref[pl.ds(sid, 1), :]
    o_ref[...] = x_ref[...] + row[None]


def kernel(x, table):
    b, s, n, d = x.shape
    enc = _renorm_table_sc(table)
    xr = x.reshape(b * s, n, d)
    out = pl.pallas_call(
        _add_enc_kernel,
        grid=(b * s,),
        in_specs=[
            pl.BlockSpec((1, n, d), lambda i: (i, 0, 0)),
            pl.BlockSpec((SEQ, d), lambda i: (0, 0)),
        ],
        out_specs=pl.BlockSpec((1, n, d), lambda i: (i, 0, 0)),
        out_shape=jax.ShapeDtypeStruct((b * s, n, d), x.dtype),
        compiler_params=pltpu.CompilerParams(
            dimension_semantics=("arbitrary",),
            vmem_limit_bytes=60 * 1024 * 1024,
        ),
    )(xr, enc)
    return out.reshape(b, s, n, d)


# speedup vs baseline: 1.4745x; 1.0010x over previous
import jax
import jax.numpy as jnp
from jax import lax
from jax.experimental import pallas as pl
from jax.experimental.pallas import tpu as pltpu
from jax.experimental.pallas import tpu_sc as plsc

SEQ = 9
DM = 1024
EPS = 1e-7
_NC = 2


def _renorm_body(table_hbm, enc_hbm, row_v, sq_v):
    wid = lax.axis_index("c") * 16 + lax.axis_index("s")

    @pl.when(wid < SEQ)
    def _():
        pltpu.sync_copy(table_hbm.at[wid], row_v)

        def sumsq(j, acc):
            v = row_v[pl.ds(j * 16, 16)]
            return acc + v * v

        acc = lax.fori_loop(0, DM // 16, sumsq, jnp.zeros((16,), jnp.float32))
        sq_v[...] = acc
        def lanesum(j, tot):
            return tot + plsc.load_gather(sq_v, [jnp.full((16,), j, jnp.int32)])

        nsq = lax.fori_loop(0, 16, lanesum, jnp.zeros((16,), jnp.float32))
        i = plsc.bitcast(nsq, jnp.int32)
        y = plsc.bitcast(jnp.int32(0x5F3759DF) - (i >> 1), jnp.float32)
        for _ in range(4):
            y = y * (1.5 - 0.5 * nsq * y * y)
        norm = nsq * y
        scale = jnp.where(norm > 1.0, 1.0 / (norm + EPS), jnp.float32(1.0))

        def scale_row(j, c):
            sl = pl.ds(j * 16, 16)
            row_v[sl] = row_v[sl] * scale
            return c

        lax.fori_loop(0, DM // 16, scale_row, 0)
        pltpu.sync_copy(row_v, enc_hbm.at[wid])


def _renorm_table_sc(table):
    return pl.kernel(
        _renorm_body,
        out_type=jax.ShapeDtypeStruct((SEQ, DM), jnp.float32),
        mesh=plsc.VectorSubcoreMesh(
            core_axis_name="c", subcore_axis_name="s", num_cores=1
        ),
        scratch_types=[
            pltpu.VMEM((DM,), jnp.float32),
            pltpu.VMEM((16,), jnp.float32),
        ],
        compiler_params=pltpu.CompilerParams(
            needs_layout_passes=False,
            skip_device_barrier=True,
        ),
    )(table)


def _add_enc_kernel(x_ref, enc_ref, o_ref):
    sid = lax.rem(pl.program_id(0), SEQ)
    row = enc_ref[pl.ds(sid, 1), :]
    o_ref[...] = x_ref[...] + row[None]


def kernel(x, table):
    b, s, n, d = x.shape
    enc = _renorm_table_sc(table)
    xr = x.reshape(b * s, n, d)
    out = pl.pallas_call(
        _add_enc_kernel,
        grid=(b * s,),
        in_specs=[
            pl.BlockSpec((1, n, d), lambda i: (i, 0, 0)),
            pl.BlockSpec((SEQ, d), lambda i: (0, 0)),
        ],
        out_specs=pl.BlockSpec((1, n, d), lambda i: (i, 0, 0)),
        out_shape=jax.ShapeDtypeStruct((b * s, n, d), x.dtype),
        compiler_params=pltpu.CompilerParams(
            dimension_semantics=("arbitrary",),
            skip_device_barrier=True,
            vmem_limit_bytes=60 * 1024 * 1024,
        ),
    )(xr, enc)
    return out.reshape(b, s, n, d)
